# baseline (speedup 1.0000x reference)
import dataclasses
import functools

import jax
import jax.numpy as jnp
from jax import lax
from jax.experimental import pallas as pl
from jax.experimental.pallas import tpu as pltpu
from jax.experimental.pallas import tpu_sc as plsc

N_U = 25000
N = 50000
D = 64
DH = 32
E = 800000
K = 128
SK = 1
SKE = SK * K
NS = 16
WB = 3128
WB_LAST = N - (NS - 1) * WB

NJ = 393
NC_PACK = 6336
E_PAD = NC_PACK * SKE

_mesh = plsc.VectorSubcoreMesh(core_axis_name="c", subcore_axis_name="s")

_cp = pltpu.CompilerParams()
for _f, _v in (("needs_layout_passes", False), ("use_tc_tiling_on_sc", False)):
    if _f in pltpu.CompilerParams.__dataclass_fields__:
        _cp = dataclasses.replace(_cp, **{_f: _v})


@functools.partial(
    pl.kernel,
    out_type=jax.ShapeDtypeStruct((2, N, DH), jnp.float32),
    mesh=_mesh,
    compiler_params=_cp,
    scratch_types=[
        pltpu.VMEM_SHARED((N, DH), jnp.float32),
        pltpu.VMEM((3, 3, SKE), jnp.int32),
        pltpu.VMEM((3, SKE, 64), jnp.float32),
        pltpu.VMEM((3, SKE), jnp.int32),
        pltpu.SemaphoreType.DMA,
        pltpu.SemaphoreType.DMA,
        pltpu.SemaphoreType.DMA,
        pltpu.SemaphoreType.DMA,
        pltpu.SemaphoreType.DMA,
        pltpu.SemaphoreType.DMA,
        pltpu.SemaphoreType.DMA,
        pltpu.SemaphoreType.DMA,
        pltpu.SemaphoreType.DMA,
    ],
)
def _layer(tbl_hbm, pack_hbm, zeros_hbm, out_hbm,
           acc, packv, rows, sidx,
           sp0, sp1, sp2, sg0, sg1, sg2, ss0, ss1, ss2):
    c = lax.axis_index("c")
    s = lax.axis_index("s")
    sems_p = (sp0, sp1, sp2)
    sems_g = (sg0, sg1, sg2)
    sems_s = (ss0, ss1, ss2)

    @pl.when(s < NS - 1)
    def _z_main():
        pltpu.sync_copy(zeros_hbm, acc.at[pl.ds(s * WB, WB)])

    @pl.when(s == NS - 1)
    def _z_last():
        pltpu.sync_copy(zeros_hbm.at[pl.ds(0, WB_LAST)],
                        acc.at[pl.ds((NS - 1) * WB, WB_LAST)])

    plsc.subcore_barrier()

    def pack_dma(j, b):
        return pltpu.make_async_copy(
            pack_hbm.at[s + NS * j], packv.at[b], sems_p[b])

    def gather_dma(b):
        return pltpu.make_async_copy(
            tbl_hbm.at[c].at[packv.at[b, 0]],
            rows.at[b], sems_g[b])

    def scatter_dma(b):
        return pltpu.make_async_copy(
            rows.at[b], acc.at[sidx.at[b]], sems_s[b])

    def compute(b):
        @pl.loop(0, SKE, step=16)
        def _si(q):
            sidx[b, pl.ds(q, 16)] = packv[b, 1, pl.ds(q, 16)]


    pack_dma(0, 0).start()
    pack_dma(1, 1).start()
    pack_dma(2, 2).start()
    pack_dma(0, 0).wait()
    gather_dma(0).start()
    pack_dma(1, 1).wait()
    gather_dma(1).start()

    @pl.loop(0, NJ, step=3)
    def _pipe(j):
        for t in range(3):
            a = t
            cc = (t + 2) % 3
            jt = j + t
            pack_dma(jt + 2, cc).wait()
            gather_dma(cc).start()
            gather_dma(a).wait()
            compute(a)
            pack_dma(jt + 3, a).start()

    gather_dma(0).wait()
    gather_dma(1).wait()
    pack_dma(NJ + 2, 2).wait()

    plsc.subcore_barrier()

    @pl.when(s < NS - 1)
    def _wb_main():
        r0 = pl.multiple_of(s * WB, 8)
        pltpu.sync_copy(acc.at[pl.ds(r0, WB)], out_hbm.at[c].at[pl.ds(r0, WB)])

    @pl.when(s == NS - 1)
    def _wb_last():
        pltpu.sync_copy(acc.at[pl.ds((NS - 1) * WB, WB_LAST)],
                        out_hbm.at[c].at[pl.ds((NS - 1) * WB, WB_LAST)])


def _combine_body(a_ref, b_ref, c_ref, d_ref, o_ref):
    o_ref[...] = (a_ref[...] + b_ref[...] + c_ref[...] + d_ref[...]) * 0.25


_combine = pl.pallas_call(
    _combine_body,
    grid=(25,),
    in_specs=[pl.BlockSpec((1000, 128), lambda i: (i, 0))] * 4,
    out_specs=pl.BlockSpec((1000, 128), lambda i: (i, 0)),
    out_shape=jax.ShapeDtypeStruct((25000, 128), jnp.float32),
)


def kernel(edge_index, edge_weight, user_emb, item_emb):
    dst = edge_index[0]
    src = edge_index[1]
    pad = E_PAD - E
    zpad = jnp.zeros((pad,), jnp.int32)
    srcp = jnp.concatenate([(src.astype(jnp.int32) // 2), zpad])
    dstp = jnp.concatenate([dst.astype(jnp.int32), zpad])
    wp = jnp.concatenate([lax.bitcast_convert_type(edge_weight, jnp.int32), zpad])
    pack = jnp.stack([srcp.reshape(NC_PACK, SKE), dstp.reshape(NC_PACK, SKE),
                      wp.reshape(NC_PACK, SKE)], axis=1)

    ego0 = jnp.concatenate([user_emb, item_emb], axis=0)
    t0 = jnp.stack([ego0[:, :DH], ego0[:, DH:]])
    zeros = jnp.zeros((WB, DH), jnp.float32)
    t1 = _layer(t0.reshape(2, 25000, 64), pack, zeros)
    t2 = _layer(t1.reshape(2, 25000, 64), pack, zeros)
    t3 = _layer(t2.reshape(2, 25000, 64), pack, zeros)
    mean_flat = _combine(t0.reshape(25000, 128), t1.reshape(25000, 128),
                         t2.reshape(25000, 128), t3.reshape(25000, 128))
    mean_split = mean_flat.reshape(2, N, DH)
    mean_emb = jnp.concatenate([mean_split[0], mean_split[1]], axis=1)
    return mean_emb[:N_U], mean_emb[N_U:]


# speedup vs baseline: 1.0985x; 1.0985x over previous
import dataclasses
import functools

import jax
import jax.numpy as jnp
from jax import lax
from jax.experimental import pallas as pl
from jax.experimental.pallas import tpu as pltpu
from jax.experimental.pallas import tpu_sc as plsc

N_U = 25000
N = 50000
D = 64
DH = 32
E = 800000
K = 128
SKE = 272
NS = 16
WB = 3128
WB_LAST = N - (NS - 1) * WB

NJ = 186
NC_PACK = 3024
E_PAD = NC_PACK * SKE

_mesh = plsc.VectorSubcoreMesh(core_axis_name="c", subcore_axis_name="s")

_cp = pltpu.CompilerParams()
for _f, _v in (("needs_layout_passes", False), ("use_tc_tiling_on_sc", False)):
    if _f in pltpu.CompilerParams.__dataclass_fields__:
        _cp = dataclasses.replace(_cp, **{_f: _v})


@functools.partial(
    pl.kernel,
    out_type=jax.ShapeDtypeStruct((2, N, DH), jnp.float32),
    mesh=_mesh,
    compiler_params=_cp,
    scratch_types=[
        pltpu.VMEM_SHARED((N, DH), jnp.float32),
        pltpu.VMEM((3, 3, SKE), jnp.int32),
        pltpu.VMEM((3, SKE, DH), jnp.float32),
        pltpu.VMEM((3, SKE), jnp.int32),
        pltpu.SemaphoreType.DMA,
        pltpu.SemaphoreType.DMA,
        pltpu.SemaphoreType.DMA,
        pltpu.SemaphoreType.DMA,
        pltpu.SemaphoreType.DMA,
        pltpu.SemaphoreType.DMA,
        pltpu.SemaphoreType.DMA,
        pltpu.SemaphoreType.DMA,
        pltpu.SemaphoreType.DMA,
    ],
)
def _layer(tbl_hbm, pack_hbm, zeros_hbm, out_hbm,
           acc, packv, rows, sidx,
           sp0, sp1, sp2, sg0, sg1, sg2, ss0, ss1, ss2):
    c = lax.axis_index("c")
    s = lax.axis_index("s")
    sems_p = (sp0, sp1, sp2)
    sems_g = (sg0, sg1, sg2)
    sems_s = (ss0, ss1, ss2)

    @pl.when(s < NS - 1)
    def _z_main():
        pltpu.sync_copy(zeros_hbm, acc.at[pl.ds(s * WB, WB)])

    @pl.when(s == NS - 1)
    def _z_last():
        pltpu.sync_copy(zeros_hbm.at[pl.ds(0, WB_LAST)],
                        acc.at[pl.ds((NS - 1) * WB, WB_LAST)])

    plsc.subcore_barrier()

    def pack_dma(j, b):
        return pltpu.make_async_copy(
            pack_hbm.at[s + NS * j], packv.at[b], sems_p[b])

    def gather_dma(b):
        return pltpu.make_async_copy(
            tbl_hbm.at[c].at[packv.at[b, 0]],
            rows.at[b], sems_g[b])

    def scatter_dma(b):
        return pltpu.make_async_copy(
            rows.at[b], acc.at[sidx.at[b]], sems_s[b])

    def compute(b):
        @pl.loop(0, SKE, step=16)
        def _si(q):
            sidx[b, pl.ds(q, 16)] = packv[b, 1, pl.ds(q, 16)]

        @pl.loop(0, SKE, step=16)
        def _sk(k0):
            wv = plsc.bitcast(packv[b, 2, pl.ds(k0, 16)], jnp.float32)
            for dk in range(16):
                wk = jnp.take_along_axis(
                    wv, jnp.full((16,), dk, jnp.int32), axis=0,
                    mode="promise_in_bounds")
                r0 = rows[b, k0 + dk, pl.ds(0, 16)]
                r1 = rows[b, k0 + dk, pl.ds(16, 16)]
                rows[b, k0 + dk, pl.ds(0, 16)] = r0 * wk
                rows[b, k0 + dk, pl.ds(16, 16)] = r1 * wk

        scatter_dma(b).start(add=True)

    pack_dma(0, 0).start()
    pack_dma(1, 1).start()
    pack_dma(2, 2).start()
    pack_dma(0, 0).wait()
    gather_dma(0).start()
    pack_dma(1, 1).wait()
    gather_dma(1).start()

    @pl.loop(0, NJ, step=3)
    def _pipe(j):
        for t in range(3):
            a = t
            cc = (t + 2) % 3
            jt = j + t
            pack_dma(jt + 2, cc).wait()
            if t == 0:
                @pl.when(j > 0)
                def _drain0():
                    scatter_dma(cc).wait()
            else:
                scatter_dma(cc).wait()
            gather_dma(cc).start()
            gather_dma(a).wait()
            compute(a)
            pack_dma(jt + 3, a).start()

    gather_dma(0).wait()
    gather_dma(1).wait()
    scatter_dma(2).wait()
    pack_dma(NJ + 2, 2).wait()

    plsc.subcore_barrier()

    @pl.when(s < NS - 1)
    def _wb_main():
        r0 = pl.multiple_of(s * WB, 8)
        pltpu.sync_copy(acc.at[pl.ds(r0, WB)], out_hbm.at[c].at[pl.ds(r0, WB)])

    @pl.when(s == NS - 1)
    def _wb_last():
        pltpu.sync_copy(acc.at[pl.ds((NS - 1) * WB, WB_LAST)],
                        out_hbm.at[c].at[pl.ds((NS - 1) * WB, WB_LAST)])


def _partial_body(a_ref, b_ref, c_ref, o_ref):
    o_ref[...] = a_ref[...] + b_ref[...] + c_ref[...]


_partial = pl.pallas_call(
    _partial_body,
    grid=(25,),
    in_specs=[pl.BlockSpec((1000, 128), lambda i: (i, 0))] * 3,
    out_specs=pl.BlockSpec((1000, 128), lambda i: (i, 0)),
    out_shape=jax.ShapeDtypeStruct((25000, 128), jnp.float32),
)


def _combine_body(a_ref, b_ref, o_ref):
    o_ref[...] = (a_ref[...] + b_ref[...]) * 0.25


_combine = pl.pallas_call(
    _combine_body,
    grid=(25,),
    in_specs=[pl.BlockSpec((1000, 128), lambda i: (i, 0))] * 2,
    out_specs=pl.BlockSpec((1000, 128), lambda i: (i, 0)),
    out_shape=jax.ShapeDtypeStruct((25000, 128), jnp.float32),
)


def kernel(edge_index, edge_weight, user_emb, item_emb):
    dst = edge_index[0]
    src = edge_index[1]
    pad = E_PAD - E
    zpad = jnp.zeros((pad,), jnp.int32)
    srcp = jnp.concatenate([src.astype(jnp.int32), zpad])
    dstp = jnp.concatenate([dst.astype(jnp.int32), zpad])
    wp = jnp.concatenate([lax.bitcast_convert_type(edge_weight, jnp.int32), zpad])
    pack = jnp.stack([srcp.reshape(NC_PACK, SKE), dstp.reshape(NC_PACK, SKE),
                      wp.reshape(NC_PACK, SKE)], axis=1)

    ego0 = jnp.concatenate([user_emb, item_emb], axis=0)
    t0 = jnp.stack([ego0[:, :DH], ego0[:, DH:]])
    zeros = jnp.zeros((WB, DH), jnp.float32)
    t1 = _layer(t0, pack, zeros)
    t2 = _layer(t1, pack, zeros)
    t3 = _layer(t2, pack, zeros)
    part = _partial(t0.reshape(25000, 128), t1.reshape(25000, 128),
                    t2.reshape(25000, 128))
    mean_flat = _combine(part, t3.reshape(25000, 128))
    mean_split = mean_flat.reshape(2, N, DH)
    mean_emb = jnp.concatenate([mean_split[0], mean_split[1]], axis=1)
    return mean_emb[:N_U], mean_emb[N_U:]


# speedup vs baseline: 1.1119x; 1.0122x over previous
import dataclasses
import functools

import jax
import jax.numpy as jnp
from jax import lax
from jax.experimental import pallas as pl
from jax.experimental.pallas import tpu as pltpu
from jax.experimental.pallas import tpu_sc as plsc

N_U = 25000
N = 50000
D = 64
DH = 32
E = 800000
K = 128
SKE = 256
NS = 16
WB = 3128
WB_LAST = N - (NS - 1) * WB

NJ = 198
NC_PACK = 3216
E_PAD = NC_PACK * SKE

_mesh = plsc.VectorSubcoreMesh(core_axis_name="c", subcore_axis_name="s")

_cp = pltpu.CompilerParams()
for _f, _v in (("needs_layout_passes", False), ("use_tc_tiling_on_sc", False)):
    if _f in pltpu.CompilerParams.__dataclass_fields__:
        _cp = dataclasses.replace(_cp, **{_f: _v})


@functools.partial(
    pl.kernel,
    out_type=jax.ShapeDtypeStruct((2, N, DH), jnp.float32),
    mesh=_mesh,
    compiler_params=_cp,
    scratch_types=[
        pltpu.VMEM_SHARED((N, DH), jnp.float32),
        pltpu.VMEM((3, 3, SKE), jnp.int32),
        pltpu.VMEM((3, SKE, DH), jnp.float32),
        pltpu.VMEM((3, SKE), jnp.int32),
        pltpu.SemaphoreType.DMA,
        pltpu.SemaphoreType.DMA,
        pltpu.SemaphoreType.DMA,
        pltpu.SemaphoreType.DMA,
        pltpu.SemaphoreType.DMA,
        pltpu.SemaphoreType.DMA,
        pltpu.SemaphoreType.DMA,
        pltpu.SemaphoreType.DMA,
        pltpu.SemaphoreType.DMA,
    ],
)
def _layer(tbl_hbm, pack_hbm, zeros_hbm, out_hbm,
           acc, packv, rows, sidx,
           sp0, sp1, sp2, sg0, sg1, sg2, ss0, ss1, ss2):
    c = lax.axis_index("c")
    s = lax.axis_index("s")
    sems_p = (sp0, sp1, sp2)
    sems_g = (sg0, sg1, sg2)
    sems_s = (ss0, ss1, ss2)

    @pl.when(s < NS - 1)
    def _z_main():
        pltpu.sync_copy(zeros_hbm, acc.at[pl.ds(s * WB, WB)])

    @pl.when(s == NS - 1)
    def _z_last():
        pltpu.sync_copy(zeros_hbm.at[pl.ds(0, WB_LAST)],
                        acc.at[pl.ds((NS - 1) * WB, WB_LAST)])

    plsc.subcore_barrier()

    def pack_dma(j, b):
        return pltpu.make_async_copy(
            pack_hbm.at[s + NS * j], packv.at[b], sems_p[b])

    def gather_dma(b):
        return pltpu.make_async_copy(
            tbl_hbm.at[c].at[packv.at[b, 0]],
            rows.at[b], sems_g[b])

    def scatter_dma(b):
        return pltpu.make_async_copy(
            rows.at[b], acc.at[sidx.at[b]], sems_s[b])

    def compute(b):
        @pl.loop(0, SKE, step=16)
        def _si(q):
            sidx[b, pl.ds(q, 16)] = packv[b, 1, pl.ds(q, 16)]

        @pl.loop(0, SKE, step=16)
        def _sk(k0):
            wv = plsc.bitcast(packv[b, 2, pl.ds(k0, 16)], jnp.float32)
            for dk in range(16):
                wk = jnp.take_along_axis(
                    wv, jnp.full((16,), dk, jnp.int32), axis=0,
                    mode="promise_in_bounds")
                r0 = rows[b, k0 + dk, pl.ds(0, 16)]
                r1 = rows[b, k0 + dk, pl.ds(16, 16)]
                rows[b, k0 + dk, pl.ds(0, 16)] = r0 * wk
                rows[b, k0 + dk, pl.ds(16, 16)] = r1 * wk

        scatter_dma(b).start(add=True)

    pack_dma(0, 0).start()
    pack_dma(1, 1).start()
    pack_dma(2, 2).start()
    pack_dma(0, 0).wait()
    gather_dma(0).start()
    pack_dma(1, 1).wait()
    gather_dma(1).start()

    @pl.loop(0, NJ, step=3)
    def _pipe(j):
        for t in range(3):
            a = t
            cc = (t + 2) % 3
            jt = j + t
            pack_dma(jt + 2, cc).wait()
            if t == 0:
                @pl.when(j > 0)
                def _drain0():
                    scatter_dma(cc).wait()
            else:
                scatter_dma(cc).wait()
            gather_dma(cc).start()
            gather_dma(a).wait()
            compute(a)
            pack_dma(jt + 3, a).start()

    gather_dma(0).wait()
    gather_dma(1).wait()
    scatter_dma(2).wait()
    pack_dma(NJ + 2, 2).wait()

    plsc.subcore_barrier()

    @pl.when(s < NS - 1)
    def _wb_main():
        r0 = pl.multiple_of(s * WB, 8)
        pltpu.sync_copy(acc.at[pl.ds(r0, WB)], out_hbm.at[c].at[pl.ds(r0, WB)])

    @pl.when(s == NS - 1)
    def _wb_last():
        pltpu.sync_copy(acc.at[pl.ds((NS - 1) * WB, WB_LAST)],
                        out_hbm.at[c].at[pl.ds((NS - 1) * WB, WB_LAST)])


def _partial_body(a_ref, b_ref, c_ref, o_ref):
    o_ref[...] = a_ref[...] + b_ref[...] + c_ref[...]


_partial = pl.pallas_call(
    _partial_body,
    grid=(25,),
    in_specs=[pl.BlockSpec((1000, 128), lambda i: (i, 0))] * 3,
    out_specs=pl.BlockSpec((1000, 128), lambda i: (i, 0)),
    out_shape=jax.ShapeDtypeStruct((25000, 128), jnp.float32),
)


def _combine_body(a_ref, b_ref, o_ref):
    o_ref[...] = (a_ref[...] + b_ref[...]) * 0.25


_combine = pl.pallas_call(
    _combine_body,
    grid=(25,),
    in_specs=[pl.BlockSpec((1000, 128), lambda i: (i, 0))] * 2,
    out_specs=pl.BlockSpec((1000, 128), lambda i: (i, 0)),
    out_shape=jax.ShapeDtypeStruct((25000, 128), jnp.float32),
)


def kernel(edge_index, edge_weight, user_emb, item_emb):
    dst = edge_index[0]
    src = edge_index[1]
    pad = E_PAD - E
    zpad = jnp.zeros((pad,), jnp.int32)
    srcp = jnp.concatenate([src.astype(jnp.int32), zpad])
    dstp = jnp.concatenate([dst.astype(jnp.int32), zpad])
    wp = jnp.concatenate([lax.bitcast_convert_type(edge_weight, jnp.int32), zpad])
    pack = jnp.stack([srcp.reshape(NC_PACK, SKE), dstp.reshape(NC_PACK, SKE),
                      wp.reshape(NC_PACK, SKE)], axis=1)

    ego0 = jnp.concatenate([user_emb, item_emb], axis=0)
    t0 = jnp.stack([ego0[:, :DH], ego0[:, DH:]])
    zeros = jnp.zeros((WB, DH), jnp.float32)
    t1 = _layer(t0, pack, zeros)
    t2 = _layer(t1, pack, zeros)
    t3 = _layer(t2, pack, zeros)
    part = _partial(t0.reshape(25000, 128), t1.reshape(25000, 128),
                    t2.reshape(25000, 128))
    mean_flat = _combine(part, t3.reshape(25000, 128))
    mean_split = mean_flat.reshape(2, N, DH)
    mean_emb = jnp.concatenate([mean_split[0], mean_split[1]], axis=1)
    return mean_emb[:N_U], mean_emb[N_U:]
